# baseline (speedup 1.0000x reference)
import functools

import jax
import jax.numpy as jnp
from jax import lax
from jax.experimental import pallas as pl
from jax.experimental.pallas import tpu as pltpu
from jax.experimental.pallas import tpu_sc as plsc

_TEMPERATURE = 0.5
_FC = 128
_TCH = 256


def _routing_idx(logits):
    key = jax.random.key(42)
    u = jax.random.uniform(
        key, logits.shape, minval=0.0, maxval=1.0, dtype=jnp.float32
    )
    g = -jnp.log(-jnp.log(u + 1e-20) + 1e-20)
    y = jax.nn.softmax((logits + g) / _TEMPERATURE, axis=-1)
    C = logits.shape[2]
    clusters = jnp.arange(C, dtype=jnp.float32)
    soft = jnp.sum(y * clusters, axis=2, keepdims=True)
    return jax.lax.stop_gradient(soft).astype(jnp.int32)[..., 0]


def _make_sc_scatter(B, D, F, C):
    NFC = F // _FC
    NCH = D // _TCH
    NG = _FC // 16
    mesh = plsc.VectorSubcoreMesh(core_axis_name="c", subcore_axis_name="s")

    @functools.partial(
        pl.kernel,
        out_type=jax.ShapeDtypeStruct((B, C, F), jnp.float32),
        mesh=mesh,
        scratch_types=[
            pltpu.VMEM((_TCH, _FC), jnp.float32),
            pltpu.VMEM((_TCH, 16), jnp.int32),
            pltpu.VMEM((C, _FC), jnp.float32),
        ],
        compiler_params=pltpu.CompilerParams(needs_layout_passes=False),
    )
    def k(values_hbm, idx_hbm, out_hbm, vbuf, ibuf, acc):
        wid = lax.axis_index("s") * 2 + lax.axis_index("c")
        b = wid // NFC
        fc = wid % NFC
        zero16 = jnp.zeros((16,), jnp.float32)

        @pl.loop(0, C)
        def _zero(r):
            for g in range(NG):
                acc[r, pl.ds(g * 16, 16)] = zero16

        cols = [jnp.arange(16, dtype=jnp.int32) + g * 16 for g in range(NG)]

        @pl.loop(0, NCH)
        def _chunk(ch):
            t0 = ch * _TCH
            pltpu.sync_copy(
                values_hbm.at[b, pl.ds(t0, _TCH), pl.ds(fc * _FC, _FC)], vbuf
            )
            pltpu.sync_copy(idx_hbm.at[b, pl.ds(t0, _TCH), :], ibuf)

            @pl.loop(0, _TCH)
            def _tok(t):
                rowv = ibuf[t, :]
                for g in range(NG):
                    x = vbuf[t, pl.ds(g * 16, 16)]
                    plsc.addupdate_scatter(acc, [rowv, cols[g]], x)

        pltpu.sync_copy(acc, out_hbm.at[b, :, pl.ds(fc * _FC, _FC)])

    return k


def kernel(values, logits):
    B, D, F = values.shape
    C = logits.shape[2]
    idx = _routing_idx(logits)
    idx16 = jnp.broadcast_to(idx[..., None], (B, D, 16))
    sc = _make_sc_scatter(B, D, F, C)
    return sc(values, idx16)


# speedup vs baseline: 1.3491x; 1.3491x over previous
import functools

import jax
import jax.numpy as jnp
from jax import lax
from jax.experimental import pallas as pl
from jax.experimental.pallas import tpu as pltpu
from jax.experimental.pallas import tpu_sc as plsc

_TEMPERATURE = 0.5
_FC = 128
_TCH = 128


def _routing_idx(logits):
    key = jax.random.key(42)
    u = jax.random.uniform(
        key, logits.shape, minval=0.0, maxval=1.0, dtype=jnp.float32
    )
    g = -jnp.log(-jnp.log(u + 1e-20) + 1e-20)
    y = jax.nn.softmax((logits + g) / _TEMPERATURE, axis=-1)
    C = logits.shape[2]
    clusters = jnp.arange(C, dtype=jnp.float32)
    soft = jnp.sum(y * clusters, axis=2, keepdims=True)
    return jax.lax.stop_gradient(soft).astype(jnp.int32)[..., 0]


def _make_sc_scatter(B, D, F, C):
    NFC = F // _FC
    NCH = D // _TCH
    NG = _FC // 16
    mesh = plsc.VectorSubcoreMesh(core_axis_name="c", subcore_axis_name="s")

    @functools.partial(
        pl.kernel,
        out_type=jax.ShapeDtypeStruct((B, C, F), jnp.float32),
        mesh=mesh,
        scratch_types=[
            pltpu.VMEM((2, _TCH, _FC), jnp.float32),
            pltpu.VMEM((2, _TCH, 16), jnp.int32),
            pltpu.VMEM((C, _FC), jnp.float32),
            pltpu.SemaphoreType.DMA,
            pltpu.SemaphoreType.DMA,
            pltpu.SemaphoreType.DMA,
            pltpu.SemaphoreType.DMA,
        ],
        compiler_params=pltpu.CompilerParams(needs_layout_passes=False),
    )
    def k(values_hbm, idx_hbm, out_hbm, vbuf, ibuf, acc, vs0, vs1, is0, is1):
        wid = lax.axis_index("s") * 2 + lax.axis_index("c")
        b = wid // NFC
        fc = wid % NFC
        vsems = (vs0, vs1)
        isems = (is0, is1)
        zero16 = jnp.zeros((16,), jnp.float32)

        @pl.loop(0, C)
        def _zero(r):
            for g in range(NG):
                acc[r, pl.ds(g * 16, 16)] = zero16

        cols = [jnp.arange(16, dtype=jnp.int32) + g * 16 for g in range(NG)]

        def _vcopy(ch, slot):
            return pltpu.make_async_copy(
                values_hbm.at[b, pl.ds(ch * _TCH, _TCH), pl.ds(fc * _FC, _FC)],
                vbuf.at[slot],
                vsems[slot],
            )

        def _icopy(ch, slot):
            return pltpu.make_async_copy(
                idx_hbm.at[b, pl.ds(ch * _TCH, _TCH), :], ibuf.at[slot], isems[slot]
            )

        _vcopy(0, 0).start()
        _icopy(0, 0).start()

        @pl.loop(0, NCH, step=2)
        def _chunk2(ch0):
            for slot in range(2):
                ch = ch0 + slot
                nxt = ch + 1

                @pl.when(nxt < NCH)
                def _prefetch():
                    _vcopy(nxt, 1 - slot).start()
                    _icopy(nxt, 1 - slot).start()

                _vcopy(ch, slot).wait()
                _icopy(ch, slot).wait()

                @pl.loop(0, _TCH, unroll=4)
                def _tok(t):
                    rowv = ibuf[slot, t, :]
                    for g in range(NG):
                        x = vbuf[slot, t, pl.ds(g * 16, 16)]
                        plsc.addupdate_scatter(acc, [rowv, cols[g]], x)

        pltpu.sync_copy(acc, out_hbm.at[b, :, pl.ds(fc * _FC, _FC)])

    return k


def kernel(values, logits):
    B, D, F = values.shape
    C = logits.shape[2]
    idx = _routing_idx(logits)
    idx16 = jnp.broadcast_to(idx[..., None], (B, D, 16))
    sc = _make_sc_scatter(B, D, F, C)
    return sc(values, idx16)


# speedup vs baseline: 3.1215x; 2.3139x over previous
import functools

import jax
import jax.numpy as jnp
from jax import lax
from jax.experimental import pallas as pl
from jax.experimental.pallas import tpu as pltpu
from jax.experimental.pallas import tpu_sc as plsc

_TEMPERATURE = 0.5
_FC = 128
_TCH = 128


def _routing_idx(logits):
    key = jax.random.key(42)
    u = jax.random.uniform(
        key, logits.shape, minval=0.0, maxval=1.0, dtype=jnp.float32
    )
    g = -jnp.log(-jnp.log(u + 1e-20) + 1e-20)
    y = jax.nn.softmax((logits + g) / _TEMPERATURE, axis=-1)
    C = logits.shape[2]
    clusters = jnp.arange(C, dtype=jnp.float32)
    soft = jnp.sum(y * clusters, axis=2, keepdims=True)
    return jax.lax.stop_gradient(soft).astype(jnp.int32)[..., 0]


def _make_sc_scatter(B, D, F, C):
    NFC = F // _FC
    NCH = D // _TCH
    NG = _FC // 16
    mesh = plsc.VectorSubcoreMesh(core_axis_name="c", subcore_axis_name="s")

    @functools.partial(
        pl.kernel,
        out_type=jax.ShapeDtypeStruct((B, C, F), jnp.float32),
        mesh=mesh,
        scratch_types=[
            pltpu.VMEM((2, _TCH, _FC), jnp.float32),
            pltpu.VMEM((2, _TCH), jnp.int32),
            pltpu.VMEM((C, _FC), jnp.float32),
            pltpu.VMEM_SHARED((16, C, _FC), jnp.float32),
            pltpu.SemaphoreType.DMA,
            pltpu.SemaphoreType.DMA,
            pltpu.SemaphoreType.DMA,
            pltpu.SemaphoreType.DMA,
            pltpu.SemaphoreType.DMA,
        ],
        compiler_params=pltpu.CompilerParams(needs_layout_passes=False),
    )
    def k(values_hbm, idx_hbm, out_hbm, vbuf, ibuf, acc, accsh, vs0, vs1, is0, is1, ss):
        sid = lax.axis_index("s")
        wid = sid * 2 + lax.axis_index("c")
        b = wid // NFC
        fc = wid % NFC
        vsems = (vs0, vs1)
        isems = (is0, is1)
        zero16 = jnp.zeros((16,), jnp.float32)

        @pl.loop(0, C)
        def _zero(r):
            for g in range(NG):
                acc[r, pl.ds(g * 16, 16)] = zero16

        def _vcopy(ch, slot):
            return pltpu.make_async_copy(
                values_hbm.at[b, pl.ds(ch * _TCH, _TCH), pl.ds(fc * _FC, _FC)],
                vbuf.at[slot],
                vsems[slot],
            )

        def _icopy(ch, slot):
            return pltpu.make_async_copy(
                idx_hbm.at[b, pl.ds(ch * _TCH, _TCH)], ibuf.at[slot], isems[slot]
            )

        def _scatter(slot):
            return pltpu.async_copy(
                vbuf.at[slot], accsh.at[sid].at[ibuf.at[slot]], ss, add=True
            )

        pltpu.sync_copy(acc, accsh.at[sid])
        _vcopy(0, 0).start()
        _icopy(0, 0).start()

        @pl.loop(0, NCH, step=2)
        def _chunk2(ch0):
            for slot in range(2):
                ch = ch0 + slot
                nxt = ch + 1

                @pl.when(nxt < NCH)
                def _prefetch():
                    _vcopy(nxt, 1 - slot).start()
                    _icopy(nxt, 1 - slot).start()

                _vcopy(ch, slot).wait()
                _icopy(ch, slot).wait()
                _scatter(slot).wait()

        pltpu.sync_copy(accsh.at[sid], acc)
        pltpu.sync_copy(acc, out_hbm.at[b, :, pl.ds(fc * _FC, _FC)])

    return k


def kernel(values, logits):
    B, D, F = values.shape
    C = logits.shape[2]
    idx = _routing_idx(logits)
    sc = _make_sc_scatter(B, D, F, C)
    return sc(values, idx)
